# baseline (speedup 1.0000x reference)
import jax
import jax.numpy as jnp
from jax import lax
from jax.experimental import pallas as pl
from jax.experimental.pallas import tpu as pltpu
from jax.experimental.pallas import tpu_sc as plsc

N = 100000
E = 6400000

NC = 2
NS = 16
NW = NC * NS

BLKS = E // 128
BPW = BLKS // NW
CB = 22
NCH = BPW // CB

N_PAD = 102400
ZSL = N_PAD // NS
NPW = N_PAD // NW


def _edge_accum_body(edges_hbm, i_hbm, zeros_hbm, parts_hbm,
                     i_vmem, chunk_vmem, valp_buf, valn_buf,
                     idxp_buf, idxn_buf, accum_sh, sem_in, sem_s):
    cid = lax.axis_index("c")
    sid = lax.axis_index("s")
    wid = cid * NS + sid

    pltpu.sync_copy(zeros_hbm.at[pl.ds(sid * ZSL, ZSL)],
                    accum_sh.at[pl.ds(sid * ZSL, ZSL)])
    pltpu.sync_copy(i_hbm, i_vmem)
    plsc.subcore_barrier()

    extra = jnp.where(wid < BLKS - BPW * NW, 1, 0)
    start = wid * BPW + jnp.where(extra == 1, wid, BLKS - BPW * NW)

    pltpu.sync_copy(edges_hbm.at[pl.ds(start, CB)], chunk_vmem.at[0])

    def chunk_body(c, carry):
        par = lax.rem(c, 2)
        nxt = lax.rem(c + 1, 2)
        b0n = jnp.where(c + 1 < NCH, start + (c + 1) * CB, start)
        din = pltpu.async_copy(edges_hbm.at[pl.ds(b0n, CB)],
                               chunk_vmem.at[nxt], sem_in)
        for b in range(CB):
            for k in range(8):
                o = b * 128 + k * 16
                src16 = chunk_vmem[par, b, 0, pl.ds(k * 16, 16)]
                dst16 = chunk_vmem[par, b, 1, pl.ds(k * 16, 16)]
                gs = plsc.load_gather(i_vmem, [src16])
                gd = plsc.load_gather(i_vmem, [dst16])
                diff = gd - gs
                valp_buf[pl.ds(o, 16)] = diff
                valn_buf[pl.ds(o, 16)] = -diff
                idxp_buf[pl.ds(o, 16)] = src16
                idxn_buf[pl.ds(o, 16)] = dst16
        dp = pltpu.async_copy(valp_buf, accum_sh.at[idxp_buf],
                              sem_s, add=True)
        dn = pltpu.async_copy(valn_buf, accum_sh.at[idxn_buf],
                              sem_s, add=True)
        dp.wait()
        dn.wait()
        din.wait()
        return carry

    lax.fori_loop(0, NCH, chunk_body, 0)

    @pl.when(extra == 1)
    def _():
        pltpu.sync_copy(edges_hbm.at[pl.ds(start + BPW, 1)],
                        chunk_vmem.at[0, pl.ds(0, 1)])
        for k in range(8):
            src16 = chunk_vmem[0, 0, 0, pl.ds(k * 16, 16)]
            dst16 = chunk_vmem[0, 0, 1, pl.ds(k * 16, 16)]
            gs = plsc.load_gather(i_vmem, [src16])
            gd = plsc.load_gather(i_vmem, [dst16])
            diff = gd - gs
            valp_buf[pl.ds(k * 16, 16)] = diff
            valn_buf[pl.ds(k * 16, 16)] = -diff
        pltpu.sync_copy(valp_buf.at[pl.ds(0, 128)],
                        accum_sh.at[chunk_vmem.at[0, 0, 0]], add=True)
        pltpu.sync_copy(valn_buf.at[pl.ds(0, 128)],
                        accum_sh.at[chunk_vmem.at[0, 0, 1]], add=True)

    plsc.subcore_barrier()
    pltpu.sync_copy(accum_sh.at[pl.ds(sid * ZSL, ZSL)],
                    parts_hbm.at[pl.ds(cid * N_PAD + sid * ZSL, ZSL)])


def _finale_body(s0_hbm, s1_hbm, b_hbm, k_hbm, parts_hbm, ib_hbm,
                 d0_hbm, d1_hbm, d2_hbm,
                 s0_buf, s1_buf, b_buf, k_buf, p0_buf, p1_buf, ib_buf,
                 d0_buf, d1_buf, d2_buf):
    cid = lax.axis_index("c")
    sid = lax.axis_index("s")
    wid = cid * NS + sid
    base = wid * NPW

    pltpu.sync_copy(s0_hbm.at[pl.ds(base, NPW)], s0_buf)
    pltpu.sync_copy(s1_hbm.at[pl.ds(base, NPW)], s1_buf)
    pltpu.sync_copy(b_hbm.at[pl.ds(base, NPW)], b_buf)
    pltpu.sync_copy(k_hbm.at[pl.ds(base, NPW)], k_buf)
    pltpu.sync_copy(parts_hbm.at[pl.ds(base, NPW)], p0_buf)
    pltpu.sync_copy(parts_hbm.at[pl.ds(N_PAD + base, NPW)], p1_buf)
    pltpu.sync_copy(ib_hbm, ib_buf)

    ibv = ib_buf[...]

    def step(i, carry):
        sl = pl.ds(i * 16, 16)
        s0 = s0_buf[sl]
        s1 = s1_buf[sl]
        b = b_buf[sl]
        k = k_buf[sl]
        p = p0_buf[sl] + p1_buf[sl]
        sb = s0 * s1 * b
        sk = s1 * k
        d0_buf[sl] = -sb
        d1_buf[sl] = sb - sk + ibv * p
        d2_buf[sl] = sk
        return carry

    lax.fori_loop(0, NPW // 16, step, 0)
    pltpu.sync_copy(d0_buf, d0_hbm.at[pl.ds(base, NPW)])
    pltpu.sync_copy(d1_buf, d1_hbm.at[pl.ds(base, NPW)])
    pltpu.sync_copy(d2_buf, d2_hbm.at[pl.ds(base, NPW)])


_MESH = plsc.VectorSubcoreMesh(core_axis_name="c", subcore_axis_name="s")
_PARAMS = pltpu.CompilerParams(needs_layout_passes=False)

_edge_accum = pl.kernel(
    _edge_accum_body,
    out_type=jax.ShapeDtypeStruct((NC * N_PAD,), jnp.float32),
    mesh=_MESH,
    compiler_params=_PARAMS,
    scratch_types=[
        pltpu.VMEM((N,), jnp.float32),
        pltpu.VMEM((2, CB, 2, 128), jnp.int32),
        pltpu.VMEM((CB * 128,), jnp.float32),
        pltpu.VMEM((CB * 128,), jnp.float32),
        pltpu.VMEM((CB * 128,), jnp.int32),
        pltpu.VMEM((CB * 128,), jnp.int32),
        pltpu.VMEM_SHARED((N_PAD,), jnp.float32),
        pltpu.SemaphoreType.DMA,
        pltpu.SemaphoreType.DMA,
    ],
)

_finale = pl.kernel(
    _finale_body,
    out_type=[jax.ShapeDtypeStruct((N_PAD,), jnp.float32)] * 3,
    mesh=_MESH,
    compiler_params=_PARAMS,
    scratch_types=(
        [pltpu.VMEM((NPW,), jnp.float32)] * 6
        + [pltpu.VMEM((16,), jnp.float32)]
        + [pltpu.VMEM((NPW,), jnp.float32)] * 3
    ),
)


@jax.jit
def kernel(t, s, intra_b, intra_k, inter_adj, inter_b):
    del t
    i_vec = s[:, 1]
    zeros = jnp.zeros((N_PAD,), jnp.float32)
    adj_blk = inter_adj.reshape(BLKS, 128, 2).transpose(0, 2, 1)
    parts = _edge_accum(adj_blk, i_vec, zeros)
    s0_pad = jnp.pad(s[:, 0], (0, N_PAD - N))
    s1_pad = jnp.pad(i_vec, (0, N_PAD - N))
    b_pad = jnp.pad(intra_b, (0, N_PAD - N))
    k_pad = jnp.pad(intra_k, (0, N_PAD - N))
    ib16 = jnp.broadcast_to(inter_b, (16,)).astype(jnp.float32)
    d0, d1, d2 = _finale(s0_pad, s1_pad, b_pad, k_pad, parts, ib16)
    return jnp.stack([d0[:N], d1[:N], d2[:N]], axis=1)


# speedup vs baseline: 1.3764x; 1.3764x over previous
import jax
import jax.numpy as jnp
from jax import lax
from jax.experimental import pallas as pl
from jax.experimental.pallas import tpu as pltpu
from jax.experimental.pallas import tpu_sc as plsc

N = 100000
E = 6400000

NC = 2
NS = 16
NW = NC * NS

BLKS = E // 128
BPW = BLKS // NW
CB = 22
NCH = BPW // CB

N_PAD = 102400
ZSL = N_PAD // NS
NPW = N_PAD // NW


def _edge_accum_body(edges_hbm, i_hbm, zeros_hbm, parts_hbm,
                     i_vmem, chunk_vmem, valp_buf, valn_buf,
                     accum_sh, sem_in, sem_s):
    cid = lax.axis_index("c")
    sid = lax.axis_index("s")
    wid = cid * NS + sid

    pltpu.sync_copy(zeros_hbm.at[pl.ds(sid * ZSL, ZSL)],
                    accum_sh.at[pl.ds(sid * ZSL, ZSL)])
    pltpu.sync_copy(i_hbm, i_vmem)
    plsc.subcore_barrier()

    extra = jnp.where(wid < BLKS - BPW * NW, 1, 0)
    start = wid * BPW + jnp.where(extra == 1, wid, BLKS - BPW * NW)

    pltpu.sync_copy(edges_hbm.at[pl.ds(start, CB)], chunk_vmem.at[0])

    def chunk_body(c, carry):
        par = lax.rem(c, 2)
        nxt = lax.rem(c + 1, 2)
        b0n = jnp.where(c + 1 < NCH, start + (c + 1) * CB, start)
        din = pltpu.async_copy(edges_hbm.at[pl.ds(b0n, CB)],
                               chunk_vmem.at[nxt], sem_in)
        descs = []
        for b in range(CB):
            for k in range(8):
                src16 = chunk_vmem[par, b, 0, pl.ds(k * 16, 16)]
                dst16 = chunk_vmem[par, b, 1, pl.ds(k * 16, 16)]
                gs = plsc.load_gather(i_vmem, [src16])
                gd = plsc.load_gather(i_vmem, [dst16])
                diff = gd - gs
                valp_buf[b, pl.ds(k * 16, 16)] = diff
                valn_buf[b, pl.ds(k * 16, 16)] = -diff
            descs.append(pltpu.async_copy(
                valp_buf.at[b], accum_sh.at[chunk_vmem.at[par, b, 0]],
                sem_s, add=True))
            descs.append(pltpu.async_copy(
                valn_buf.at[b], accum_sh.at[chunk_vmem.at[par, b, 1]],
                sem_s, add=True))
        for d in descs:
            d.wait()
        din.wait()
        return carry

    lax.fori_loop(0, NCH, chunk_body, 0)

    @pl.when(extra == 1)
    def _():
        pltpu.sync_copy(edges_hbm.at[pl.ds(start + BPW, 1)],
                        chunk_vmem.at[0, pl.ds(0, 1)])
        for k in range(8):
            src16 = chunk_vmem[0, 0, 0, pl.ds(k * 16, 16)]
            dst16 = chunk_vmem[0, 0, 1, pl.ds(k * 16, 16)]
            gs = plsc.load_gather(i_vmem, [src16])
            gd = plsc.load_gather(i_vmem, [dst16])
            diff = gd - gs
            valp_buf[0, pl.ds(k * 16, 16)] = diff
            valn_buf[0, pl.ds(k * 16, 16)] = -diff
        pltpu.sync_copy(valp_buf.at[0],
                        accum_sh.at[chunk_vmem.at[0, 0, 0]], add=True)
        pltpu.sync_copy(valn_buf.at[0],
                        accum_sh.at[chunk_vmem.at[0, 0, 1]], add=True)

    plsc.subcore_barrier()
    pltpu.sync_copy(accum_sh.at[pl.ds(sid * ZSL, ZSL)],
                    parts_hbm.at[pl.ds(cid * N_PAD + sid * ZSL, ZSL)])


def _finale_body(s0_hbm, s1_hbm, b_hbm, k_hbm, parts_hbm, ib_hbm,
                 d0_hbm, d1_hbm, d2_hbm,
                 s0_buf, s1_buf, b_buf, k_buf, p0_buf, p1_buf, ib_buf,
                 d0_buf, d1_buf, d2_buf):
    cid = lax.axis_index("c")
    sid = lax.axis_index("s")
    wid = cid * NS + sid
    base = wid * NPW

    pltpu.sync_copy(s0_hbm.at[pl.ds(base, NPW)], s0_buf)
    pltpu.sync_copy(s1_hbm.at[pl.ds(base, NPW)], s1_buf)
    pltpu.sync_copy(b_hbm.at[pl.ds(base, NPW)], b_buf)
    pltpu.sync_copy(k_hbm.at[pl.ds(base, NPW)], k_buf)
    pltpu.sync_copy(parts_hbm.at[pl.ds(base, NPW)], p0_buf)
    pltpu.sync_copy(parts_hbm.at[pl.ds(N_PAD + base, NPW)], p1_buf)
    pltpu.sync_copy(ib_hbm, ib_buf)

    ibv = ib_buf[...]

    def step(i, carry):
        sl = pl.ds(i * 16, 16)
        s0 = s0_buf[sl]
        s1 = s1_buf[sl]
        b = b_buf[sl]
        k = k_buf[sl]
        p = p0_buf[sl] + p1_buf[sl]
        sb = s0 * s1 * b
        sk = s1 * k
        d0_buf[sl] = -sb
        d1_buf[sl] = sb - sk + ibv * p
        d2_buf[sl] = sk
        return carry

    lax.fori_loop(0, NPW // 16, step, 0)
    pltpu.sync_copy(d0_buf, d0_hbm.at[pl.ds(base, NPW)])
    pltpu.sync_copy(d1_buf, d1_hbm.at[pl.ds(base, NPW)])
    pltpu.sync_copy(d2_buf, d2_hbm.at[pl.ds(base, NPW)])


_MESH = plsc.VectorSubcoreMesh(core_axis_name="c", subcore_axis_name="s")
_PARAMS = pltpu.CompilerParams(needs_layout_passes=False)

_edge_accum = pl.kernel(
    _edge_accum_body,
    out_type=jax.ShapeDtypeStruct((NC * N_PAD,), jnp.float32),
    mesh=_MESH,
    compiler_params=_PARAMS,
    scratch_types=[
        pltpu.VMEM((N,), jnp.float32),
        pltpu.VMEM((2, CB, 2, 128), jnp.int32),
        pltpu.VMEM((CB, 128), jnp.float32),
        pltpu.VMEM((CB, 128), jnp.float32),
        pltpu.VMEM_SHARED((N_PAD,), jnp.float32),
        pltpu.SemaphoreType.DMA,
        pltpu.SemaphoreType.DMA,
    ],
)

_finale = pl.kernel(
    _finale_body,
    out_type=[jax.ShapeDtypeStruct((N_PAD,), jnp.float32)] * 3,
    mesh=_MESH,
    compiler_params=_PARAMS,
    scratch_types=(
        [pltpu.VMEM((NPW,), jnp.float32)] * 6
        + [pltpu.VMEM((16,), jnp.float32)]
        + [pltpu.VMEM((NPW,), jnp.float32)] * 3
    ),
)


@jax.jit
def kernel(t, s, intra_b, intra_k, inter_adj, inter_b):
    del t
    i_vec = s[:, 1]
    zeros = jnp.zeros((N_PAD,), jnp.float32)
    adj_blk = inter_adj.reshape(BLKS, 128, 2).transpose(0, 2, 1)
    parts = _edge_accum(adj_blk, i_vec, zeros)
    s0_pad = jnp.pad(s[:, 0], (0, N_PAD - N))
    s1_pad = jnp.pad(i_vec, (0, N_PAD - N))
    b_pad = jnp.pad(intra_b, (0, N_PAD - N))
    k_pad = jnp.pad(intra_k, (0, N_PAD - N))
    ib16 = jnp.broadcast_to(inter_b, (16,)).astype(jnp.float32)
    d0, d1, d2 = _finale(s0_pad, s1_pad, b_pad, k_pad, parts, ib16)
    return jnp.stack([d0[:N], d1[:N], d2[:N]], axis=1)


# speedup vs baseline: 1.5826x; 1.1498x over previous
import jax
import jax.numpy as jnp
from jax import lax
from jax.experimental import pallas as pl
from jax.experimental.pallas import tpu as pltpu
from jax.experimental.pallas import tpu_sc as plsc

N = 100000
E = 6400000

NC = 2
NS = 16
NW = NC * NS

BLKS = E // 128
BPW = BLKS // NW
CB = 11
NCH = BPW // CB

N_PAD = 102400
ZSL = N_PAD // NS
NPW = N_PAD // NW


def _edge_accum_body(edges_hbm, i_hbm, zeros_hbm, parts_hbm,
                     i_vmem, chunk_vmem, valp_buf, valn_buf,
                     accum_sh, sem_in, sem_s):
    cid = lax.axis_index("c")
    sid = lax.axis_index("s")
    wid = cid * NS + sid

    pltpu.sync_copy(zeros_hbm.at[pl.ds(sid * ZSL, ZSL)],
                    accum_sh.at[pl.ds(sid * ZSL, ZSL)])
    pltpu.sync_copy(i_hbm, i_vmem)
    plsc.subcore_barrier()

    extra = jnp.where(wid < BLKS - BPW * NW, 1, 0)
    start = wid * BPW + jnp.where(extra == 1, wid, BLKS - BPW * NW)

    pltpu.sync_copy(edges_hbm.at[pl.ds(start, CB)], chunk_vmem.at[0])

    def chunk_body(c, carry):
        par = lax.rem(c, 2)
        nxt = lax.rem(c + 1, 2)
        b0n = jnp.where(c + 1 < NCH, start + (c + 1) * CB, start)
        din = pltpu.async_copy(edges_hbm.at[pl.ds(b0n, CB)],
                               chunk_vmem.at[nxt], sem_in)
        descs = []
        for b in range(CB):
            for k in range(8):
                src16 = chunk_vmem[par, b, 0, pl.ds(k * 16, 16)]
                dst16 = chunk_vmem[par, b, 1, pl.ds(k * 16, 16)]
                gs = plsc.load_gather(i_vmem, [src16])
                gd = plsc.load_gather(i_vmem, [dst16])
                diff = gd - gs
                valp_buf[b, pl.ds(k * 16, 16)] = diff
                valn_buf[b, pl.ds(k * 16, 16)] = -diff
            descs.append(pltpu.async_copy(
                valp_buf.at[b], accum_sh.at[chunk_vmem.at[par, b, 0]],
                sem_s, add=True))
            descs.append(pltpu.async_copy(
                valn_buf.at[b], accum_sh.at[chunk_vmem.at[par, b, 1]],
                sem_s, add=True))
        for d in descs:
            d.wait()
        din.wait()
        return carry

    lax.fori_loop(0, NCH, chunk_body, 0)

    @pl.when(extra == 1)
    def _():
        pltpu.sync_copy(edges_hbm.at[pl.ds(start + BPW, 1)],
                        chunk_vmem.at[0, pl.ds(0, 1)])
        for k in range(8):
            src16 = chunk_vmem[0, 0, 0, pl.ds(k * 16, 16)]
            dst16 = chunk_vmem[0, 0, 1, pl.ds(k * 16, 16)]
            gs = plsc.load_gather(i_vmem, [src16])
            gd = plsc.load_gather(i_vmem, [dst16])
            diff = gd - gs
            valp_buf[0, pl.ds(k * 16, 16)] = diff
            valn_buf[0, pl.ds(k * 16, 16)] = -diff
        pltpu.sync_copy(valp_buf.at[0],
                        accum_sh.at[chunk_vmem.at[0, 0, 0]], add=True)
        pltpu.sync_copy(valn_buf.at[0],
                        accum_sh.at[chunk_vmem.at[0, 0, 1]], add=True)

    plsc.subcore_barrier()
    pltpu.sync_copy(accum_sh.at[pl.ds(sid * ZSL, ZSL)],
                    parts_hbm.at[pl.ds(cid * N_PAD + sid * ZSL, ZSL)])


def _finale_body(s0_hbm, s1_hbm, b_hbm, k_hbm, parts_hbm, ib_hbm,
                 d0_hbm, d1_hbm, d2_hbm,
                 s0_buf, s1_buf, b_buf, k_buf, p0_buf, p1_buf, ib_buf,
                 d0_buf, d1_buf, d2_buf):
    cid = lax.axis_index("c")
    sid = lax.axis_index("s")
    wid = cid * NS + sid
    base = wid * NPW

    pltpu.sync_copy(s0_hbm.at[pl.ds(base, NPW)], s0_buf)
    pltpu.sync_copy(s1_hbm.at[pl.ds(base, NPW)], s1_buf)
    pltpu.sync_copy(b_hbm.at[pl.ds(base, NPW)], b_buf)
    pltpu.sync_copy(k_hbm.at[pl.ds(base, NPW)], k_buf)
    pltpu.sync_copy(parts_hbm.at[pl.ds(base, NPW)], p0_buf)
    pltpu.sync_copy(parts_hbm.at[pl.ds(N_PAD + base, NPW)], p1_buf)
    pltpu.sync_copy(ib_hbm, ib_buf)

    ibv = ib_buf[...]

    def step(i, carry):
        sl = pl.ds(i * 16, 16)
        s0 = s0_buf[sl]
        s1 = s1_buf[sl]
        b = b_buf[sl]
        k = k_buf[sl]
        p = p0_buf[sl] + p1_buf[sl]
        sb = s0 * s1 * b
        sk = s1 * k
        d0_buf[sl] = -sb
        d1_buf[sl] = sb - sk + ibv * p
        d2_buf[sl] = sk
        return carry

    lax.fori_loop(0, NPW // 16, step, 0)
    pltpu.sync_copy(d0_buf, d0_hbm.at[pl.ds(base, NPW)])
    pltpu.sync_copy(d1_buf, d1_hbm.at[pl.ds(base, NPW)])
    pltpu.sync_copy(d2_buf, d2_hbm.at[pl.ds(base, NPW)])


_MESH = plsc.VectorSubcoreMesh(core_axis_name="c", subcore_axis_name="s")
_PARAMS = pltpu.CompilerParams(needs_layout_passes=False)

_edge_accum = pl.kernel(
    _edge_accum_body,
    out_type=jax.ShapeDtypeStruct((NC * N_PAD,), jnp.float32),
    mesh=_MESH,
    compiler_params=_PARAMS,
    scratch_types=[
        pltpu.VMEM((N,), jnp.float32),
        pltpu.VMEM((2, CB, 2, 128), jnp.int32),
        pltpu.VMEM((CB, 128), jnp.float32),
        pltpu.VMEM((CB, 128), jnp.float32),
        pltpu.VMEM_SHARED((N_PAD,), jnp.float32),
        pltpu.SemaphoreType.DMA,
        pltpu.SemaphoreType.DMA,
    ],
)

_finale = pl.kernel(
    _finale_body,
    out_type=[jax.ShapeDtypeStruct((N_PAD,), jnp.float32)] * 3,
    mesh=_MESH,
    compiler_params=_PARAMS,
    scratch_types=(
        [pltpu.VMEM((NPW,), jnp.float32)] * 6
        + [pltpu.VMEM((16,), jnp.float32)]
        + [pltpu.VMEM((NPW,), jnp.float32)] * 3
    ),
)


@jax.jit
def kernel(t, s, intra_b, intra_k, inter_adj, inter_b):
    del t
    i_vec = s[:, 1]
    zeros = jnp.zeros((N_PAD,), jnp.float32)
    adj_blk = inter_adj.reshape(BLKS, 128, 2).transpose(0, 2, 1)
    parts = _edge_accum(adj_blk, i_vec, zeros)
    s0_pad = jnp.pad(s[:, 0], (0, N_PAD - N))
    s1_pad = jnp.pad(i_vec, (0, N_PAD - N))
    b_pad = jnp.pad(intra_b, (0, N_PAD - N))
    k_pad = jnp.pad(intra_k, (0, N_PAD - N))
    ib16 = jnp.broadcast_to(inter_b, (16,)).astype(jnp.float32)
    d0, d1, d2 = _finale(s0_pad, s1_pad, b_pad, k_pad, parts, ib16)
    return jnp.stack([d0[:N], d1[:N], d2[:N]], axis=1)
